# baseline (speedup 1.0000x reference)
import jax
import jax.numpy as jnp
from jax.experimental import pallas as pl

B = 4
SEQ = 1024
H = 8
D = 64
T = B * SEQ
BLOCK_SIZE = 32
NB = SEQ // BLOCK_SIZE
S = 8
TQ = 256
NT = SEQ // TQ
BPT = TQ // BLOCK_SIZE


def _silu(x):
    return x * jax.nn.sigmoid(x)


def _body(q_ref, k_ref, v_ref, wc_ref, ws_ref, o_ref):
    qm = q_ref[0, 0]
    km = k_ref[0, 0]
    vm = v_ref[0, 0]
    scale = D ** (-0.5)

    g_cmp = jax.nn.sigmoid(jnp.sum(qm * wc_ref[0][None, :], axis=1,
                                   keepdims=True))
    g_slc = jax.nn.sigmoid(jnp.sum(qm * ws_ref[0][None, :], axis=1,
                                   keepdims=True))

    kc = jnp.mean(km.reshape(NB, BLOCK_SIZE, D), axis=1)
    vc = jnp.mean(vm.reshape(NB, BLOCK_SIZE, D), axis=1)

    sc = jax.lax.dot_general(qm, kc, (((1,), (1,)), ((), ())),
                             preferred_element_type=jnp.float32) * scale
    jcol = jax.lax.broadcasted_iota(jnp.int32, (SEQ, NB), 1)
    qblk = jax.lax.broadcasted_iota(jnp.int32, (SEQ, NB), 0) // BLOCK_SIZE
    causal_blk = qblk >= jcol
    sm = jnp.where(causal_blk, sc, -1e9)

    rank = jnp.zeros((SEQ, NB), jnp.float32)
    for kk in range(NB):
        col = sm[:, kk:kk + 1]
        beats = (col > sm) | ((col == sm) & (kk < jcol))
        rank = rank + beats.astype(jnp.float32)
    mf = ((rank < float(S)) & causal_blk).astype(jnp.float32)

    p_cmp = jnp.where(causal_blk, _silu(sc), 0.0)
    o_cmp = jnp.dot(p_cmp, vc,
                    preferred_element_type=jnp.float32) * g_cmp

    e_row = jax.lax.broadcasted_iota(jnp.int32, (BPT, TQ), 0)
    e_col = jax.lax.broadcasted_iota(jnp.int32, (BPT, TQ), 1) // BLOCK_SIZE
    E8 = (e_row == e_col).astype(jnp.float32)
    qpos = jax.lax.broadcasted_iota(jnp.int32, (TQ, TQ), 0)
    kpos = jax.lax.broadcasted_iota(jnp.int32, (TQ, TQ), 1)
    causal_tile = (qpos >= kpos).astype(jnp.float32)

    for ti in range(NT):
        qt = qm[ti * TQ:(ti + 1) * TQ]
        acc = jnp.zeros((TQ, D), jnp.float32)
        for tj in range(ti + 1):
            kt = km[tj * TQ:(tj + 1) * TQ]
            vt = vm[tj * TQ:(tj + 1) * TQ]
            s = jax.lax.dot_general(qt, kt, (((1,), (1,)), ((), ())),
                                    preferred_element_type=jnp.float32)
            s = s * scale
            mt = mf[ti * TQ:(ti + 1) * TQ, tj * BPT:(tj + 1) * BPT]
            mexp = jnp.dot(mt, E8,
                           preferred_element_type=jnp.float32)
            p = _silu(s) * mexp
            if ti == tj:
                p = p * causal_tile
            acc = acc + jnp.dot(p, vt,
                                preferred_element_type=jnp.float32)
        sl = slice(ti * TQ, (ti + 1) * TQ)
        o_ref[0, 0, sl, :] = o_cmp[sl] + acc * g_slc[sl]


def kernel(q, k, v, u, x_offsets, Wg_cmp, Wg_slc, Wg_swa):
    qh = q.reshape(B, SEQ, H, D).transpose(0, 2, 1, 3)
    kh = k.reshape(B, SEQ, H, D).transpose(0, 2, 1, 3)
    vh = v.reshape(B, SEQ, H, D).transpose(0, 2, 1, 3)

    out4 = pl.pallas_call(
        _body,
        grid=(B, H),
        in_specs=[
            pl.BlockSpec((1, 1, SEQ, D), lambda b, h: (b, h, 0, 0)),
            pl.BlockSpec((1, 1, SEQ, D), lambda b, h: (b, h, 0, 0)),
            pl.BlockSpec((1, 1, SEQ, D), lambda b, h: (b, h, 0, 0)),
            pl.BlockSpec((1, D), lambda b, h: (0, 0)),
            pl.BlockSpec((1, D), lambda b, h: (0, 0)),
        ],
        out_specs=pl.BlockSpec((1, 1, SEQ, D), lambda b, h: (b, h, 0, 0)),
        out_shape=jax.ShapeDtypeStruct((B, H, SEQ, D), jnp.float32),
    )(qh, kh, vh, Wg_cmp.reshape(1, D), Wg_slc.reshape(1, D))

    return out4.transpose(0, 2, 1, 3).reshape(T, H, D)
